# baseline (speedup 1.0000x reference)
import functools
import jax
import jax.numpy as jnp
from jax import lax
from jax.experimental import pallas as pl
from jax.experimental.pallas import tpu as pltpu
from jax.experimental.pallas import tpu_sc as plsc

D = 64
SCALE = 8.0
L = 16

_info = plsc.get_sparse_core_info()
NC, NS = _info.num_cores, _info.num_subcores
NW = NC * NS

_mesh = plsc.VectorSubcoreMesh(core_axis_name="c", subcore_axis_name="s")
_params = pltpu.CompilerParams(needs_layout_passes=False)

PBLK = 128


def _make_pairs(V):
    P = V // 2
    K_PW = P // (PBLK * NW)
    n_full = K_PW * NW
    rem_p0 = n_full * PBLK
    n_rem = P - rem_p0
    n_chunks = n_rem // 8

    @functools.partial(
        pl.kernel, mesh=_mesh,
        out_type=jax.ShapeDtypeStruct((P, 2 * D), jnp.float32),
        compiler_params=_params,
        scratch_types=[
            pltpu.VMEM((2, D, 2 * PBLK), jnp.float32),
            pltpu.VMEM((2, PBLK, 2 * D), jnp.float32),
            pltpu.VMEM((16, D), jnp.float32),
            pltpu.VMEM((8, 2 * D), jnp.float32),
            pltpu.SemaphoreType.DMA,
            pltpu.SemaphoreType.DMA,
            pltpu.SemaphoreType.DMA,
            pltpu.SemaphoreType.DMA,
        ],
    )
    def _tr(tabT_hbm, rem_hbm, out_hbm, in_v, out_v, rem_v, rout_v,
            i0, i1, o0, o1):
        isem = [i0, i1]
        osem = [o0, o1]
        wid = lax.axis_index("s") * NC + lax.axis_index("c")
        lanes = lax.broadcasted_iota(jnp.int32, (L,), 0)

        def in_desc(blk, b):
            return pltpu.make_async_copy(
                tabT_hbm.at[:, pl.ds(blk * 2 * PBLK, 2 * PBLK)],
                in_v.at[b], isem[b])

        def out_desc(blk, b):
            return pltpu.make_async_copy(
                out_v.at[b], out_hbm.at[pl.ds(blk * PBLK, PBLK)], osem[b])

        def transpose(b):
            def prow(p, _):
                for h in range(2):
                    col = 2 * p + h
                    for cc in range(D // L):
                        vals = plsc.load_gather(
                            in_v, [jnp.full((L,), b, jnp.int32),
                                   cc * L + lanes,
                                   jnp.full((L,), col, jnp.int32)])
                        out_v[b, p, pl.ds(h * D + cc * L, L)] = vals
                return 0
            lax.fori_loop(0, PBLK, prow, 0)

        def blk_of(k):
            return k * NW + wid

        in_desc(blk_of(0), 0).start()

        def outer(o, _):
            for b in range(2):
                k = o * 2 + b
                nb = 1 - b
                if True:
                    @pl.when(k + 1 < K_PW)
                    def _():
                        @pl.when(k >= 1)
                        def _():
                            out_desc(blk_of(k - 1), nb).wait()
                        in_desc(blk_of(k + 1), nb).start()
                in_desc(blk_of(k), b).wait()
                transpose(b)
                out_desc(blk_of(k), b).start()
            return 0

        lax.fori_loop(0, K_PW // 2, outer, 0)
        out_desc(blk_of(K_PW - 2), 0).wait()
        out_desc(blk_of(K_PW - 1), 1).wait()

        def rem_chunk(t, _):
            chunk = t * NW + wid
            @pl.when(chunk < n_chunks)
            def _():
                pltpu.sync_copy(rem_hbm.at[pl.ds(chunk * 16, 16)], rem_v)

                def prow(p, _):
                    for h in range(2):
                        for cc in range(D // L):
                            rout_v[p, pl.ds(h * D + cc * L, L)] = (
                                rem_v[2 * p + h, pl.ds(cc * L, L)])
                    return 0
                lax.fori_loop(0, 8, prow, 0)
                pltpu.sync_copy(
                    rout_v, out_hbm.at[pl.ds(rem_p0 + chunk * 8, 8)])
            return 0

        lax.fori_loop(0, (n_chunks + NW - 1) // NW, rem_chunk, 0)

    return _tr


def _make_gather(S, B, V):
    BW = B // NW
    assert BW % L == 0

    @functools.partial(
        pl.kernel, mesh=_mesh,
        out_type=jax.ShapeDtypeStruct((S, D, B), jnp.float32),
        compiler_params=_params,
        scratch_types=[
            pltpu.VMEM((S, BW), jnp.int32),
            pltpu.VMEM((S, BW), jnp.int32),
            pltpu.VMEM((2, BW, 2 * D), jnp.float32),
            pltpu.VMEM((2, D, BW), jnp.float32),
            pltpu.SemaphoreType.DMA,
            pltpu.SemaphoreType.DMA,
            pltpu.SemaphoreType.DMA,
            pltpu.SemaphoreType.DMA,
        ],
    )
    def _gt(xT_hbm, tab_hbm, out_hbm, idx_v, pidx_v, rows_v, outs_v,
            g0, g1, o0, o1):
        gsem = [g0, g1]
        osem = [o0, o1]
        wid = lax.axis_index("s") * NC + lax.axis_index("c")
        b0 = wid * BW
        lanes = lax.broadcasted_iota(jnp.int32, (L,), 0)

        pltpu.sync_copy(xT_hbm.at[:, pl.ds(b0, BW)], idx_v)

        def mk_pairs(i, _):
            s = i // (BW // L)
            q = i % (BW // L)
            v = idx_v[s, pl.ds(q * L, L)]
            pidx_v[s, pl.ds(q * L, L)] = jax.lax.shift_right_logical(v, 1)
            return 0

        lax.fori_loop(0, S * (BW // L), mk_pairs, 0)

        def g_desc(s, b):
            return pltpu.make_async_copy(
                tab_hbm.at[pidx_v.at[s]], rows_v.at[b], gsem[b])

        def o_desc(s, b):
            return pltpu.make_async_copy(
                outs_v.at[b], out_hbm.at[s, :, pl.ds(b0, BW)], osem[b])

        def select_scale(s, b):
            def erow(e, _):
                for c in range(BW // L):
                    iv = idx_v[s, pl.ds(c * L, L)]
                    col = jnp.bitwise_and(iv, 1) * D + e
                    vals = plsc.load_gather(
                        rows_v, [jnp.full((L,), b, jnp.int32),
                                 c * L + lanes, col])
                    outs_v[b, e, pl.ds(c * L, L)] = vals * SCALE
                return 0
            lax.fori_loop(0, D, erow, 0)

        g_desc(0, 0).start()

        def outer(o, _):
            for b in range(2):
                s = o * 2 + b
                nb = 1 - b
                @pl.when(s + 1 < S)
                def _():
                    @pl.when(s >= 1)
                    def _():
                        o_desc(s - 1, nb).wait()
                    g_desc(s + 1, nb).start()

                g_desc(s, b).wait()
                select_scale(s, b)
                o_desc(s, b).start()
            return 0

        lax.fori_loop(0, S // 2, outer, 0)
        o_desc(S - 2, 0).wait()
        o_desc(S - 1, 1).wait()

    return _gt


def kernel(x, table):
    B, S = x.shape
    V = table.shape[0]
    xT = x.T.astype(jnp.int32)
    rem = table[2 * (V // 2 // (PBLK * NW)) * PBLK * NW:]
    tab2 = _make_pairs(V)(table.T, rem)
    out = _make_gather(S, B, V)(xT, tab2)
    return jnp.transpose(out, (2, 0, 1))


# speedup vs baseline: 1.8201x; 1.8201x over previous
import functools
import jax
import jax.numpy as jnp
from jax import lax
from jax.experimental import pallas as pl
from jax.experimental.pallas import tpu as pltpu
from jax.experimental.pallas import tpu_sc as plsc

D = 64
SCALE = 8.0
L = 16

_info = plsc.get_sparse_core_info()
NC, NS = _info.num_cores, _info.num_subcores
NW = NC * NS

_mesh = plsc.VectorSubcoreMesh(core_axis_name="c", subcore_axis_name="s")
_params = pltpu.CompilerParams(needs_layout_passes=False)


def _make_gather(S, B, V):
    BW = B // NW
    assert BW % L == 0
    NCH = BW // L

    @functools.partial(
        pl.kernel, mesh=_mesh,
        out_type=jax.ShapeDtypeStruct((S, D, B), jnp.float32),
        compiler_params=_params,
        scratch_types=[
            pltpu.VMEM((S, BW), jnp.int32),
            pltpu.VMEM((S, BW), jnp.int32),
            pltpu.VMEM((2, BW, 2 * D), jnp.float32),
            pltpu.VMEM((2, D, BW), jnp.float32),
            pltpu.SemaphoreType.DMA,
            pltpu.SemaphoreType.DMA,
            pltpu.SemaphoreType.DMA,
            pltpu.SemaphoreType.DMA,
        ],
    )
    def _gt(xT_hbm, tab_hbm, out_hbm, idx_v, pidx_v, rows_v, outs_v,
            g0, g1, o0, o1):
        gsem = [g0, g1]
        osem = [o0, o1]
        wid = lax.axis_index("s") * NC + lax.axis_index("c")
        b0 = wid * BW
        lanes = lax.broadcasted_iota(jnp.int32, (L,), 0)

        pltpu.sync_copy(xT_hbm.at[:, pl.ds(b0, BW)], idx_v)

        def mk_pairs(i, _):
            s = i // NCH
            q = i % NCH
            v = idx_v[s, pl.ds(q * L, L)]
            pidx_v[s, pl.ds(q * L, L)] = jax.lax.shift_right_logical(v, 1)
            return 0

        lax.fori_loop(0, S * NCH, mk_pairs, 0)

        def g_desc(s, b):
            return pltpu.make_async_copy(
                tab_hbm.at[pidx_v.at[s]], rows_v.at[b], gsem[b])

        def o_desc(s, b):
            return pltpu.make_async_copy(
                outs_v.at[b], out_hbm.at[s, :, pl.ds(b0, BW)], osem[b])

        def select(s, b):
            cbs = []
            for c in range(NCH):
                iv = idx_v[s, pl.ds(c * L, L)]
                cbs.append(jnp.bitwise_and(iv, 1) * D)
            bvec = jnp.full((L,), b, jnp.int32)

            def erow(e, _):
                for c in range(NCH):
                    vals = plsc.load_gather(
                        rows_v, [bvec, c * L + lanes, cbs[c] + e])
                    outs_v[b, e, pl.ds(c * L, L)] = vals
                return 0
            lax.fori_loop(0, D, erow, 0)

        g_desc(0, 0).start()

        def outer(o, _):
            for b in range(2):
                s = o * 2 + b
                nb = 1 - b
                @pl.when(s + 1 < S)
                def _():
                    @pl.when(s >= 1)
                    def _():
                        o_desc(s - 1, nb).wait()
                    g_desc(s + 1, nb).start()

                g_desc(s, b).wait()
                select(s, b)
                o_desc(s, b).start()
            return 0

        lax.fori_loop(0, S // 2, outer, 0)
        o_desc(S - 2, 0).wait()
        o_desc(S - 1, 1).wait()

    return _gt


def kernel(x, table):
    B, S = x.shape
    V = table.shape[0]
    xT = x.T.astype(jnp.int32)
    tab2 = (table * SCALE).reshape(V // 2, 2 * D)
    out = _make_gather(S, B, V)(xT, tab2)
    return jnp.transpose(out, (2, 0, 1))
